# Initial kernel scaffold; baseline (speedup 1.0000x reference)
import jax
import jax.numpy as jnp
from jax import lax
from jax.experimental import pallas as pl
from jax.experimental.pallas import tpu as pltpu
from jax.experimental.pallas import tpu_sc as plsc

N_ROWS = 16384
N_COLS = 200
D = 32
B = N_ROWS * N_COLS
NW = 32
R = B // NW
C = 1024
G = C // 128
NCH = R // C


def _sc_body(x2d, table, out, idx_v, rows_v, sem):
    wid = lax.axis_index("s") * 2 + lax.axis_index("c")
    base128 = wid * (R // 128)

    def chunk(ci, carry):
        r0 = base128 + ci * G
        pltpu.sync_copy(x2d.at[pl.ds(r0, G)], idx_v)
        cps = [
            pltpu.async_copy(
                table.at[idx_v.at[j]],
                rows_v.at[pl.ds(j * 128, 128)],
                sem,
            )
            for j in range(G)
        ]
        for cp in cps:
            cp.wait()
        pltpu.sync_copy(rows_v, out.at[pl.ds(r0 * 128, C)])
        return carry

    lax.fori_loop(0, NCH, chunk, 0)


def kernel(x, table):
    xf = x.reshape(-1).astype(jnp.int32).reshape(B // 128, 128)
    mesh = plsc.VectorSubcoreMesh(core_axis_name="c", subcore_axis_name="s")
    out = pl.kernel(
        _sc_body,
        out_type=jax.ShapeDtypeStruct((B, D), jnp.float32),
        mesh=mesh,
        scratch_types=[
            pltpu.VMEM((G, 128), jnp.int32),
            pltpu.VMEM((C, D), jnp.float32),
            pltpu.SemaphoreType.DMA,
        ],
    )(xf, table)
    return out.reshape(N_ROWS, N_COLS, D)


# speedup vs baseline: 4.8071x; 4.8071x over previous
import jax
import jax.numpy as jnp
from jax import lax
from jax.experimental import pallas as pl
from jax.experimental.pallas import tpu as pltpu
from jax.experimental.pallas import tpu_sc as plsc

N_ROWS = 16384
N_COLS = 200
D = 32
B = N_ROWS * N_COLS
NW = 32
R = B // NW
C = 1024
G = C // 128
NCH = R // C


def _sc_body(x2d, table, out, idx_v, rows_v, sem):
    wid = lax.axis_index("s") * 2 + lax.axis_index("c")
    base128 = wid * (R // 128)

    def chunk(ci, carry):
        r0 = base128 + ci * G
        pltpu.sync_copy(x2d.at[pl.ds(r0, G)], idx_v)
        cps = [
            pltpu.async_copy(
                table.at[idx_v.at[j]],
                rows_v.at[pl.ds(j * 128, 128)],
                sem,
            )
            for j in range(G)
        ]
        for cp in cps:
            cp.wait()
        pltpu.sync_copy(rows_v, out.at[pl.ds(r0 * 128, C)])
        return carry

    lax.fori_loop(0, NCH, chunk, 0)


def kernel(x, table):
    xf = x.reshape(-1).astype(jnp.int32).reshape(B // 128, 128)
    mesh = plsc.VectorSubcoreMesh(core_axis_name="c", subcore_axis_name="s")
    out = pl.kernel(
        _sc_body,
        out_type=jax.ShapeDtypeStruct((B, D), jnp.float32),
        mesh=mesh,
        scratch_types=[
            pltpu.VMEM((G, 128), jnp.int32),
            pltpu.VMEM((C, D), jnp.float32),
            pltpu.SemaphoreType.DMA,
        ],
        compiler_params=pltpu.CompilerParams(use_tc_tiling_on_sc=False),
    )(xf, table)
    return out.reshape(N_ROWS, N_COLS, D)


# speedup vs baseline: 5.0345x; 1.0473x over previous
import jax
import jax.numpy as jnp
from jax import lax
from jax.experimental import pallas as pl
from jax.experimental.pallas import tpu as pltpu
from jax.experimental.pallas import tpu_sc as plsc

N_ROWS = 16384
N_COLS = 200
D = 32
B = N_ROWS * N_COLS
NW = 32
R = B // NW
C = 1024
G = C // 128
NCH = R // C


def _sc_body(x2d, table, out,
             idx0, idx1, rows0, rows1,
             si0, si1, sg0, sg1, so0, so1):
    wid = lax.axis_index("s") * 2 + lax.axis_index("c")
    base128 = wid * (R // 128)

    bufs = ((idx0, rows0, si0, sg0, so0), (idx1, rows1, si1, sg1, so1))

    def idx_start(ci, idxv, sem):
        pltpu.async_copy(x2d.at[pl.ds(base128 + ci * G, G)], idxv, sem)

    idx_start(0, idx0, si0)

    def pair(p, carry):
        for b in range(2):
            idxv, rowsv, si, sg, so = bufs[b]
            n_idxv, _, n_si, _, _ = bufs[1 - b]
            ci = 2 * p + b

            pltpu.make_async_copy(x2d.at[pl.ds(0, G)], idxv, si).wait()

            @pl.when(ci + 1 < NCH)
            def _():
                idx_start(ci + 1, n_idxv, n_si)

            @pl.when(ci >= 2)
            def _():
                pltpu.make_async_copy(rowsv, out.at[pl.ds(0, C)], so).wait()

            cps = [
                pltpu.async_copy(
                    table.at[idxv.at[j]],
                    rowsv.at[pl.ds(j * 128, 128)],
                    sg,
                )
                for j in range(G)
            ]
            for cp in cps:
                cp.wait()

            pltpu.async_copy(rowsv, out.at[pl.ds((base128 + ci * G) * 128, C)], so)
        return carry

    lax.fori_loop(0, NCH // 2, pair, 0)

    pltpu.make_async_copy(rows0, out.at[pl.ds(0, C)], so0).wait()
    pltpu.make_async_copy(rows1, out.at[pl.ds(0, C)], so1).wait()


def kernel(x, table):
    xf = x.reshape(-1).astype(jnp.int32).reshape(B // 128, 128)
    mesh = plsc.VectorSubcoreMesh(core_axis_name="c", subcore_axis_name="s")
    out = pl.kernel(
        _sc_body,
        out_type=jax.ShapeDtypeStruct((B, D), jnp.float32),
        mesh=mesh,
        scratch_types=[
            pltpu.VMEM((G, 128), jnp.int32),
            pltpu.VMEM((G, 128), jnp.int32),
            pltpu.VMEM((C, D), jnp.float32),
            pltpu.VMEM((C, D), jnp.float32),
            pltpu.SemaphoreType.DMA,
            pltpu.SemaphoreType.DMA,
            pltpu.SemaphoreType.DMA,
            pltpu.SemaphoreType.DMA,
            pltpu.SemaphoreType.DMA,
            pltpu.SemaphoreType.DMA,
        ],
        compiler_params=pltpu.CompilerParams(use_tc_tiling_on_sc=False),
    )(xf, table)
    return out.reshape(N_ROWS, N_COLS, D)
